# baseline (speedup 1.0000x reference)
import functools

import jax
import jax.numpy as jnp
from jax import lax
from jax.experimental import pallas as pl
from jax.experimental.pallas import tpu as pltpu
from jax.experimental.pallas import tpu_sc as plsc

N = 10000
E = 160000
NCORE = 2
NSUB = 16
NW = NCORE * NSUB
EPW = E // NW
CH = 125
NCH = EPW // CH
NPAD = 10240
RPS = NPAD // NSUB
RZ = 128
RB = 2000

F32 = jnp.float32


def _fill(ref, rows, cols, value):
    per_row = cols // 16

    def body(i, carry):
        r = i // per_row
        c = (i % per_row) * 16
        ref[r, pl.ds(c, 16)] = jnp.full((16,), value, F32)
        return carry

    lax.fori_loop(0, rows * per_row, body, 0)


def _deg_body(dst_hbm, out_hbm, didx, ones, zbuf, acc):
    cid = lax.axis_index("c")
    sid = lax.axis_index("s")
    wid = cid * NSUB + sid
    _fill(ones, CH, 128, 1.0)
    _fill(zbuf, RZ, 128, 0.0)
    pltpu.sync_copy(dst_hbm.at[wid], didx)
    for z in range(RPS // RZ):
        pltpu.sync_copy(zbuf, acc.at[pl.ds(sid * RPS + z * RZ, RZ)])
    plsc.subcore_barrier()
    for c in range(NCH):
        pltpu.sync_copy(ones, acc.at[didx.at[c]], add=True)
    plsc.subcore_barrier()
    pltpu.sync_copy(acc.at[pl.ds(sid * RPS, RPS)],
                    out_hbm.at[cid, pl.ds(sid * RPS, RPS)])


def _make_deg():
    mesh = plsc.VectorSubcoreMesh(core_axis_name="c", subcore_axis_name="s")
    return pl.kernel(
        _deg_body,
        out_type=jax.ShapeDtypeStruct((NCORE, NPAD, 128), F32),
        mesh=mesh,
        scratch_types=[
            pltpu.VMEM((NCH, CH), jnp.int32),
            pltpu.VMEM((CH, 128), F32),
            pltpu.VMEM((RZ, 128), F32),
            pltpu.VMEM_SHARED((NPAD, 128), F32),
        ],
    )


def _make_prop(nb):
    mesh = plsc.VectorSubcoreMesh(core_axis_name="c", subcore_axis_name="s")
    bd = 128

    def body(*refs):
        y = refs[:nb]
        src_hbm, dst_hbm = refs[nb], refs[nb + 1]
        outs = refs[nb + 2:2 * nb + 2]
        (sidx, didx, gbuf, zbuf, acc, sem) = refs[2 * nb + 2:]
        cid = lax.axis_index("c")
        sid = lax.axis_index("s")
        wid = cid * NSUB + sid
        _fill(zbuf, RZ, bd, 0.0)
        pltpu.sync_copy(src_hbm.at[wid], sidx)
        pltpu.sync_copy(dst_hbm.at[wid], didx)
        for b in range(nb):
            for z in range(RPS // RZ):
                pltpu.sync_copy(zbuf, acc.at[pl.ds(sid * RPS + z * RZ, RZ)])
            plsc.subcore_barrier()
            for c in range(NCH):
                pltpu.async_copy(y[b].at[sidx.at[c]], gbuf, sem).wait()
                pltpu.sync_copy(gbuf, acc.at[didx.at[c]], add=True)
            plsc.subcore_barrier()
            pltpu.sync_copy(acc.at[pl.ds(sid * RPS, RPS)],
                            outs[b].at[cid, pl.ds(sid * RPS, RPS)])

    return pl.kernel(
        body,
        out_type=[jax.ShapeDtypeStruct((NCORE, NPAD, bd), F32) for _ in range(nb)],
        mesh=mesh,
        scratch_types=[
            pltpu.VMEM((NCH, CH), jnp.int32),
            pltpu.VMEM((NCH, CH), jnp.int32),
            pltpu.VMEM((CH, bd), F32),
            pltpu.VMEM((RZ, bd), F32),
            pltpu.VMEM_SHARED((NPAD, bd), F32),
            pltpu.SemaphoreType.DMA,
        ],
    )


def _mm(a, b):
    return lax.dot_general(a, b, (((1,), (0,)), ((), ())),
                           precision=lax.Precision.HIGHEST,
                           preferred_element_type=F32)


def _ka_body(degp, x, dinv_o, y0_o, y1_o):
    deg = degp[0, :, 0:1] + degp[1, :, 0:1] + 1.0
    dv = lax.rsqrt(deg)
    dinv_o[...] = dv
    y = x[...] * dv
    y0_o[...] = y[:, :128]
    y1_o[...] = y[:, 128:]


def _ka(degp, x):
    return pl.pallas_call(
        _ka_body,
        grid=(N // RB,),
        in_specs=[
            pl.BlockSpec((NCORE, RB, 128), lambda i: (0, i, 0)),
            pl.BlockSpec((RB, 256), lambda i: (i, 0)),
        ],
        out_specs=[
            pl.BlockSpec((RB, 1), lambda i: (i, 0)),
            pl.BlockSpec((RB, 128), lambda i: (i, 0)),
            pl.BlockSpec((RB, 128), lambda i: (i, 0)),
        ],
        out_shape=[
            jax.ShapeDtypeStruct((N, 1), F32),
            jax.ShapeDtypeStruct((N, 128), F32),
            jax.ShapeDtypeStruct((N, 128), F32),
        ],
    )(degp, x)


def _k1_body(t10, t11, y10, y11, dinv, W1, b1, W2, o0, o1, o2, o3):
    dv = dinv[...]
    p0 = dv * (t10[0] + t10[1] + y10[...])
    p1 = dv * (t11[0] + t11[1] + y11[...])
    W1m = W1[...]
    h = _mm(p0, W1m[:128]) + _mm(p1, W1m[128:]) + b1[...]
    h = jnp.maximum(h, 0.0)
    y2 = _mm(h, W2[...]) * dv
    o0[...] = y2[:, 0:128]
    o1[...] = y2[:, 128:256]
    o2[...] = y2[:, 256:384]
    o3[...] = y2[:, 384:512]


def _k1(t10, t11, y10, y11, dinv, W1, b1, W2):
    tspec = pl.BlockSpec((NCORE, RB, 128), lambda i: (0, i, 0))
    yspec = pl.BlockSpec((RB, 128), lambda i: (i, 0))
    return pl.pallas_call(
        _k1_body,
        grid=(N // RB,),
        in_specs=[
            tspec, tspec, yspec, yspec,
            pl.BlockSpec((RB, 1), lambda i: (i, 0)),
            pl.BlockSpec((256, 512), lambda i: (0, 0)),
            pl.BlockSpec((1, 512), lambda i: (0, 0)),
            pl.BlockSpec((512, 512), lambda i: (0, 0)),
        ],
        out_specs=[yspec, yspec, yspec, yspec],
        out_shape=[jax.ShapeDtypeStruct((N, 128), F32) for _ in range(4)],
    )(t10, t11, y10, y11, dinv, W1, b1, W2)


def _k3_body(t20, t21, t22, t23, y20, y21, y22, y23, dinv, b2, W3p, out):
    dv = dinv[...]
    W3m = W3p[...]
    b2m = b2[...]
    ts = (t20, t21, t22, t23)
    ys = (y20, y21, y22, y23)
    z = jnp.zeros((RB, 128), F32)
    for b in range(4):
        q = ts[b][0] + ts[b][1] + ys[b][...]
        h = jnp.maximum(dv * q + b2m[:, b * 128:(b + 1) * 128], 0.0)
        z = z + _mm(h, W3m[b * 128:(b + 1) * 128])
    out[...] = z * dv


def _k3(t2s, y2s, dinv, b2, W3p):
    tspec = pl.BlockSpec((NCORE, RB, 128), lambda i: (0, i, 0))
    yspec = pl.BlockSpec((RB, 128), lambda i: (i, 0))
    return pl.pallas_call(
        _k3_body,
        grid=(N // RB,),
        in_specs=[
            tspec, tspec, tspec, tspec,
            yspec, yspec, yspec, yspec,
            pl.BlockSpec((RB, 1), lambda i: (i, 0)),
            pl.BlockSpec((1, 512), lambda i: (0, 0)),
            pl.BlockSpec((512, 128), lambda i: (0, 0)),
        ],
        out_specs=pl.BlockSpec((RB, 128), lambda i: (i, 0)),
        out_shape=jax.ShapeDtypeStruct((N, 128), F32),
    )(*t2s, *y2s, dinv, b2, W3p)


def _k4_body(t3, y3, dinv, b3, out):
    q = t3[0] + t3[1] + y3[...]
    out[...] = dinv[...] * q[:, :40] + b3[...]


def _k4(t3, y3, dinv, b3):
    return pl.pallas_call(
        _k4_body,
        grid=(N // RB,),
        in_specs=[
            pl.BlockSpec((NCORE, RB, 128), lambda i: (0, i, 0)),
            pl.BlockSpec((RB, 128), lambda i: (i, 0)),
            pl.BlockSpec((RB, 1), lambda i: (i, 0)),
            pl.BlockSpec((1, 40), lambda i: (0, 0)),
        ],
        out_specs=pl.BlockSpec((RB, 40), lambda i: (i, 0)),
        out_shape=jax.ShapeDtypeStruct((N, 40), F32),
    )(t3, y3, dinv, b3)


def kernel(x, edge_index, W1, b1, W2, b2, W3, b3):
    src = edge_index[0].reshape(NW, NCH, CH)
    dst = edge_index[1].reshape(NW, NCH, CH)
    W3p = jnp.pad(W3, ((0, 0), (0, 88)))

    degp = _make_deg()(dst)
    dinv, y10, y11 = _ka(degp, x)

    t10, t11 = _make_prop(2)(y10, y11, src, dst)
    y2s = _k1(t10, t11, y10, y11, dinv, W1, b1.reshape(1, -1), W2)

    t2s = _make_prop(4)(*y2s, src, dst)
    y3 = _k3(t2s, y2s, dinv, b2.reshape(1, -1), W3p)

    (t3,) = _make_prop(1)(y3, src, dst)
    return _k4(t3, y3, dinv, b3.reshape(1, -1))


# speedup vs baseline: 1.0743x; 1.0743x over previous
import functools

import jax
import jax.numpy as jnp
from jax import lax
from jax.experimental import pallas as pl
from jax.experimental.pallas import tpu as pltpu
from jax.experimental.pallas import tpu_sc as plsc

N = 10000
E = 160000
NCORE = 2
NSUB = 16
NW = NCORE * NSUB
EPW = E // NW
CH = 125
NCH = EPW // CH
NPAD = 10240
RPS = NPAD // NSUB
RZ = 128
RB = 2000

F32 = jnp.float32


def _fill(ref, rows, cols, value):
    per_row = cols // 16

    def body(i, carry):
        r = i // per_row
        c = (i % per_row) * 16
        ref[r, pl.ds(c, 16)] = jnp.full((16,), value, F32)
        return carry

    lax.fori_loop(0, rows * per_row, body, 0)


def _deg_body(dst_hbm, out_hbm, didx, ones, zbuf, acc):
    cid = lax.axis_index("c")
    sid = lax.axis_index("s")
    wid = cid * NSUB + sid
    _fill(ones, CH, 128, 1.0)
    _fill(zbuf, RZ, 128, 0.0)
    pltpu.sync_copy(dst_hbm.at[wid], didx)
    for z in range(RPS // RZ):
        pltpu.sync_copy(zbuf, acc.at[pl.ds(sid * RPS + z * RZ, RZ)])
    plsc.subcore_barrier()
    for c in range(NCH):
        pltpu.sync_copy(ones, acc.at[didx.at[c]], add=True)
    plsc.subcore_barrier()
    pltpu.sync_copy(acc.at[pl.ds(sid * RPS, RPS)],
                    out_hbm.at[cid, pl.ds(sid * RPS, RPS)])


def _make_deg():
    mesh = plsc.VectorSubcoreMesh(core_axis_name="c", subcore_axis_name="s")
    return pl.kernel(
        _deg_body,
        out_type=jax.ShapeDtypeStruct((NCORE, NPAD, 128), F32),
        mesh=mesh,
        scratch_types=[
            pltpu.VMEM((NCH, CH), jnp.int32),
            pltpu.VMEM((CH, 128), F32),
            pltpu.VMEM((RZ, 128), F32),
            pltpu.VMEM_SHARED((NPAD, 128), F32),
        ],
    )


def _make_prop(nb):
    mesh = plsc.VectorSubcoreMesh(core_axis_name="c", subcore_axis_name="s")
    bd = 128

    def body(*refs):
        y = refs[:nb]
        src_hbm, dst_hbm = refs[nb], refs[nb + 1]
        outs = refs[nb + 2:2 * nb + 2]
        (sidx, didx, gbuf, zbuf, acc, sem) = refs[2 * nb + 2:]
        cid = lax.axis_index("c")
        sid = lax.axis_index("s")
        wid = cid * NSUB + sid
        _fill(zbuf, RZ, bd, 0.0)
        pltpu.sync_copy(src_hbm.at[wid], sidx)
        pltpu.sync_copy(dst_hbm.at[wid], didx)
        for b in range(nb):
            for z in range(RPS // RZ):
                pltpu.sync_copy(zbuf, acc.at[pl.ds(sid * RPS + z * RZ, RZ)])
            plsc.subcore_barrier()
            for c in range(NCH):
                pltpu.async_copy(y[b].at[sidx.at[c]], gbuf, sem).wait()
                pltpu.sync_copy(gbuf, acc.at[didx.at[c]], add=True)
            plsc.subcore_barrier()
            pltpu.sync_copy(acc.at[pl.ds(sid * RPS, RPS)],
                            outs[b].at[cid, pl.ds(sid * RPS, RPS)])

    return pl.kernel(
        body,
        out_type=[jax.ShapeDtypeStruct((NCORE, NPAD, bd), F32) for _ in range(nb)],
        mesh=mesh,
        scratch_types=[
            pltpu.VMEM((NCH, CH), jnp.int32),
            pltpu.VMEM((NCH, CH), jnp.int32),
            pltpu.VMEM((CH, bd), F32),
            pltpu.VMEM((RZ, bd), F32),
            pltpu.VMEM_SHARED((NPAD, bd), F32),
            pltpu.SemaphoreType.DMA,
        ],
    )


def _mm(a, b):
    return lax.dot_general(a, b, (((1,), (0,)), ((), ())),
                           precision=lax.Precision.DEFAULT,
                           preferred_element_type=F32)


def _ka_body(degp, x, dinv_o, y0_o, y1_o):
    deg = degp[0, :, 0:1] + degp[1, :, 0:1] + 1.0
    dv = lax.rsqrt(deg)
    dinv_o[...] = dv
    y = x[...] * dv
    y0_o[...] = y[:, :128]
    y1_o[...] = y[:, 128:]


def _ka(degp, x):
    return pl.pallas_call(
        _ka_body,
        grid=(N // RB,),
        in_specs=[
            pl.BlockSpec((NCORE, RB, 128), lambda i: (0, i, 0)),
            pl.BlockSpec((RB, 256), lambda i: (i, 0)),
        ],
        out_specs=[
            pl.BlockSpec((RB, 1), lambda i: (i, 0)),
            pl.BlockSpec((RB, 128), lambda i: (i, 0)),
            pl.BlockSpec((RB, 128), lambda i: (i, 0)),
        ],
        out_shape=[
            jax.ShapeDtypeStruct((N, 1), F32),
            jax.ShapeDtypeStruct((N, 128), F32),
            jax.ShapeDtypeStruct((N, 128), F32),
        ],
    )(degp, x)


def _k1_body(t10, t11, y10, y11, dinv, W1, b1, W2, o0, o1, o2, o3):
    dv = dinv[...]
    p0 = dv * (t10[0] + t10[1] + y10[...])
    p1 = dv * (t11[0] + t11[1] + y11[...])
    W1m = W1[...]
    h = _mm(p0, W1m[:128]) + _mm(p1, W1m[128:]) + b1[...]
    h = jnp.maximum(h, 0.0)
    y2 = _mm(h, W2[...]) * dv
    o0[...] = y2[:, 0:128]
    o1[...] = y2[:, 128:256]
    o2[...] = y2[:, 256:384]
    o3[...] = y2[:, 384:512]


def _k1(t10, t11, y10, y11, dinv, W1, b1, W2):
    tspec = pl.BlockSpec((NCORE, RB, 128), lambda i: (0, i, 0))
    yspec = pl.BlockSpec((RB, 128), lambda i: (i, 0))
    return pl.pallas_call(
        _k1_body,
        grid=(N // RB,),
        in_specs=[
            tspec, tspec, yspec, yspec,
            pl.BlockSpec((RB, 1), lambda i: (i, 0)),
            pl.BlockSpec((256, 512), lambda i: (0, 0)),
            pl.BlockSpec((1, 512), lambda i: (0, 0)),
            pl.BlockSpec((512, 512), lambda i: (0, 0)),
        ],
        out_specs=[yspec, yspec, yspec, yspec],
        out_shape=[jax.ShapeDtypeStruct((N, 128), F32) for _ in range(4)],
    )(t10, t11, y10, y11, dinv, W1, b1, W2)


def _k3_body(t20, t21, t22, t23, y20, y21, y22, y23, dinv, b2, W3p, out):
    dv = dinv[...]
    W3m = W3p[...]
    b2m = b2[...]
    ts = (t20, t21, t22, t23)
    ys = (y20, y21, y22, y23)
    z = jnp.zeros((RB, 128), F32)
    for b in range(4):
        q = ts[b][0] + ts[b][1] + ys[b][...]
        h = jnp.maximum(dv * q + b2m[:, b * 128:(b + 1) * 128], 0.0)
        z = z + _mm(h, W3m[b * 128:(b + 1) * 128])
    out[...] = z * dv


def _k3(t2s, y2s, dinv, b2, W3p):
    tspec = pl.BlockSpec((NCORE, RB, 128), lambda i: (0, i, 0))
    yspec = pl.BlockSpec((RB, 128), lambda i: (i, 0))
    return pl.pallas_call(
        _k3_body,
        grid=(N // RB,),
        in_specs=[
            tspec, tspec, tspec, tspec,
            yspec, yspec, yspec, yspec,
            pl.BlockSpec((RB, 1), lambda i: (i, 0)),
            pl.BlockSpec((1, 512), lambda i: (0, 0)),
            pl.BlockSpec((512, 128), lambda i: (0, 0)),
        ],
        out_specs=pl.BlockSpec((RB, 128), lambda i: (i, 0)),
        out_shape=jax.ShapeDtypeStruct((N, 128), F32),
    )(*t2s, *y2s, dinv, b2, W3p)


def _k4_body(t3, y3, dinv, b3, out):
    q = t3[0] + t3[1] + y3[...]
    out[...] = dinv[...] * q[:, :40] + b3[...]


def _k4(t3, y3, dinv, b3):
    return pl.pallas_call(
        _k4_body,
        grid=(N // RB,),
        in_specs=[
            pl.BlockSpec((NCORE, RB, 128), lambda i: (0, i, 0)),
            pl.BlockSpec((RB, 128), lambda i: (i, 0)),
            pl.BlockSpec((RB, 1), lambda i: (i, 0)),
            pl.BlockSpec((1, 40), lambda i: (0, 0)),
        ],
        out_specs=pl.BlockSpec((RB, 40), lambda i: (i, 0)),
        out_shape=jax.ShapeDtypeStruct((N, 40), F32),
    )(t3, y3, dinv, b3)


def kernel(x, edge_index, W1, b1, W2, b2, W3, b3):
    src = edge_index[0].reshape(NW, NCH, CH)
    dst = edge_index[1].reshape(NW, NCH, CH)
    W3p = jnp.pad(W3, ((0, 0), (0, 88)))

    degp = _make_deg()(dst)
    dinv, y10, y11 = _ka(degp, x)

    t10, t11 = _make_prop(2)(y10, y11, src, dst)
    y2s = _k1(t10, t11, y10, y11, dinv, W1, b1.reshape(1, -1), W2)

    t2s = _make_prop(4)(*y2s, src, dst)
    y3 = _k3(t2s, y2s, dinv, b2.reshape(1, -1), W3p)

    (t3,) = _make_prop(1)(y3, src, dst)
    return _k4(t3, y3, dinv, b3.reshape(1, -1))


# speedup vs baseline: 1.0864x; 1.0113x over previous
import functools

import jax
import jax.numpy as jnp
from jax import lax
from jax.experimental import pallas as pl
from jax.experimental.pallas import tpu as pltpu
from jax.experimental.pallas import tpu_sc as plsc

N = 10000
E = 160000
NCORE = 2
NSUB = 16
NW = NCORE * NSUB
EPW = E // NW
CH = 125
NCH = EPW // CH
NPAD = 10240
RPS = NPAD // NSUB
RZ = 128
RB = 2000

F32 = jnp.float32


def _fill(ref, rows, cols, value):
    per_row = cols // 16

    def body(i, carry):
        r = i // per_row
        c = (i % per_row) * 16
        ref[r, pl.ds(c, 16)] = jnp.full((16,), value, F32)
        return carry

    lax.fori_loop(0, rows * per_row, body, 0)


def _deg_body(dst_hbm, out_hbm, didx, ones, zbuf, acc):
    cid = lax.axis_index("c")
    sid = lax.axis_index("s")
    wid = cid * NSUB + sid
    _fill(ones, CH, 128, 1.0)
    _fill(zbuf, RZ, 128, 0.0)
    pltpu.sync_copy(dst_hbm.at[wid], didx)
    for z in range(RPS // RZ):
        pltpu.sync_copy(zbuf, acc.at[pl.ds(sid * RPS + z * RZ, RZ)])
    plsc.subcore_barrier()
    for c in range(NCH):
        pltpu.sync_copy(ones, acc.at[didx.at[c]], add=True)
    plsc.subcore_barrier()
    pltpu.sync_copy(acc.at[pl.ds(sid * RPS, RPS)],
                    out_hbm.at[cid, pl.ds(sid * RPS, RPS)])


def _make_deg():
    mesh = plsc.VectorSubcoreMesh(core_axis_name="c", subcore_axis_name="s")
    return pl.kernel(
        _deg_body,
        out_type=jax.ShapeDtypeStruct((NCORE, NPAD, 128), F32),
        mesh=mesh,
        scratch_types=[
            pltpu.VMEM((NCH, CH), jnp.int32),
            pltpu.VMEM((CH, 128), F32),
            pltpu.VMEM((RZ, 128), F32),
            pltpu.VMEM_SHARED((NPAD, 128), F32),
        ],
    )


def _make_prop(nb):
    mesh = plsc.VectorSubcoreMesh(core_axis_name="c", subcore_axis_name="s")
    bd = 128

    def body(*refs):
        y = refs[:nb]
        src_hbm, dst_hbm = refs[nb], refs[nb + 1]
        outs = refs[nb + 2:2 * nb + 2]
        (sidx, didx, gbuf, zbuf, acc, sem, lsem) = refs[2 * nb + 2:]
        cid = lax.axis_index("c")
        sid = lax.axis_index("s")
        wid = cid * NSUB + sid
        _fill(zbuf, RZ, bd, 0.0)
        pltpu.sync_copy(src_hbm.at[wid], sidx)
        pltpu.sync_copy(dst_hbm.at[wid], didx)
        gd = pltpu.async_copy(y[0].at[sidx.at[0]], gbuf, sem)
        for b in range(nb):
            dz = [pltpu.async_copy(zbuf,
                                   acc.at[pl.ds(sid * RPS + z * RZ, RZ)], lsem)
                  for z in range(RPS // RZ)]
            for d in dz:
                d.wait()
            plsc.subcore_barrier()
            for c in range(NCH):
                gd.wait()
                pltpu.sync_copy(gbuf, acc.at[didx.at[c]], add=True)
                if c + 1 < NCH:
                    gd = pltpu.async_copy(y[b].at[sidx.at[c + 1]], gbuf, sem)
                elif b + 1 < nb:
                    gd = pltpu.async_copy(y[b + 1].at[sidx.at[0]], gbuf, sem)
            plsc.subcore_barrier()
            dd = [pltpu.async_copy(acc.at[pl.ds(sid * RPS + z * RZ, RZ)],
                                   outs[b].at[cid,
                                              pl.ds(sid * RPS + z * RZ, RZ)],
                                   lsem)
                  for z in range(RPS // RZ)]
            for d in dd:
                d.wait()

    return pl.kernel(
        body,
        out_type=[jax.ShapeDtypeStruct((NCORE, NPAD, bd), F32) for _ in range(nb)],
        mesh=mesh,
        scratch_types=[
            pltpu.VMEM((NCH, CH), jnp.int32),
            pltpu.VMEM((NCH, CH), jnp.int32),
            pltpu.VMEM((CH, bd), F32),
            pltpu.VMEM((RZ, bd), F32),
            pltpu.VMEM_SHARED((NPAD, bd), F32),
            pltpu.SemaphoreType.DMA,
            pltpu.SemaphoreType.DMA,
        ],
    )


def _mm(a, b):
    return lax.dot_general(a, b, (((1,), (0,)), ((), ())),
                           precision=lax.Precision.DEFAULT,
                           preferred_element_type=F32)


def _ka_body(degp, x, dinv_o, y0_o, y1_o):
    deg = degp[0, :, 0:1] + degp[1, :, 0:1] + 1.0
    dv = lax.rsqrt(deg)
    dinv_o[...] = dv
    y = x[...] * dv
    y0_o[...] = y[:, :128]
    y1_o[...] = y[:, 128:]


def _ka(degp, x):
    return pl.pallas_call(
        _ka_body,
        grid=(N // RB,),
        in_specs=[
            pl.BlockSpec((NCORE, RB, 128), lambda i: (0, i, 0)),
            pl.BlockSpec((RB, 256), lambda i: (i, 0)),
        ],
        out_specs=[
            pl.BlockSpec((RB, 1), lambda i: (i, 0)),
            pl.BlockSpec((RB, 128), lambda i: (i, 0)),
            pl.BlockSpec((RB, 128), lambda i: (i, 0)),
        ],
        out_shape=[
            jax.ShapeDtypeStruct((N, 1), F32),
            jax.ShapeDtypeStruct((N, 128), F32),
            jax.ShapeDtypeStruct((N, 128), F32),
        ],
    )(degp, x)


def _k1_body(t10, t11, y10, y11, dinv, W1, b1, W2, o0, o1, o2, o3):
    dv = dinv[...]
    p0 = dv * (t10[0] + t10[1] + y10[...])
    p1 = dv * (t11[0] + t11[1] + y11[...])
    W1m = W1[...]
    h = _mm(p0, W1m[:128]) + _mm(p1, W1m[128:]) + b1[...]
    h = jnp.maximum(h, 0.0)
    y2 = _mm(h, W2[...]) * dv
    o0[...] = y2[:, 0:128]
    o1[...] = y2[:, 128:256]
    o2[...] = y2[:, 256:384]
    o3[...] = y2[:, 384:512]


def _k1(t10, t11, y10, y11, dinv, W1, b1, W2):
    tspec = pl.BlockSpec((NCORE, RB, 128), lambda i: (0, i, 0))
    yspec = pl.BlockSpec((RB, 128), lambda i: (i, 0))
    return pl.pallas_call(
        _k1_body,
        grid=(N // RB,),
        in_specs=[
            tspec, tspec, yspec, yspec,
            pl.BlockSpec((RB, 1), lambda i: (i, 0)),
            pl.BlockSpec((256, 512), lambda i: (0, 0)),
            pl.BlockSpec((1, 512), lambda i: (0, 0)),
            pl.BlockSpec((512, 512), lambda i: (0, 0)),
        ],
        out_specs=[yspec, yspec, yspec, yspec],
        out_shape=[jax.ShapeDtypeStruct((N, 128), F32) for _ in range(4)],
    )(t10, t11, y10, y11, dinv, W1, b1, W2)


def _k3_body(t20, t21, t22, t23, y20, y21, y22, y23, dinv, b2, W3p, out):
    dv = dinv[...]
    W3m = W3p[...]
    b2m = b2[...]
    ts = (t20, t21, t22, t23)
    ys = (y20, y21, y22, y23)
    z = jnp.zeros((RB, 128), F32)
    for b in range(4):
        q = ts[b][0] + ts[b][1] + ys[b][...]
        h = jnp.maximum(dv * q + b2m[:, b * 128:(b + 1) * 128], 0.0)
        z = z + _mm(h, W3m[b * 128:(b + 1) * 128])
    out[...] = z * dv


def _k3(t2s, y2s, dinv, b2, W3p):
    tspec = pl.BlockSpec((NCORE, RB, 128), lambda i: (0, i, 0))
    yspec = pl.BlockSpec((RB, 128), lambda i: (i, 0))
    return pl.pallas_call(
        _k3_body,
        grid=(N // RB,),
        in_specs=[
            tspec, tspec, tspec, tspec,
            yspec, yspec, yspec, yspec,
            pl.BlockSpec((RB, 1), lambda i: (i, 0)),
            pl.BlockSpec((1, 512), lambda i: (0, 0)),
            pl.BlockSpec((512, 128), lambda i: (0, 0)),
        ],
        out_specs=pl.BlockSpec((RB, 128), lambda i: (i, 0)),
        out_shape=jax.ShapeDtypeStruct((N, 128), F32),
    )(*t2s, *y2s, dinv, b2, W3p)


def _k4_body(t3, y3, dinv, b3, out):
    q = t3[0] + t3[1] + y3[...]
    out[...] = dinv[...] * q[:, :40] + b3[...]


def _k4(t3, y3, dinv, b3):
    return pl.pallas_call(
        _k4_body,
        grid=(N // RB,),
        in_specs=[
            pl.BlockSpec((NCORE, RB, 128), lambda i: (0, i, 0)),
            pl.BlockSpec((RB, 128), lambda i: (i, 0)),
            pl.BlockSpec((RB, 1), lambda i: (i, 0)),
            pl.BlockSpec((1, 40), lambda i: (0, 0)),
        ],
        out_specs=pl.BlockSpec((RB, 40), lambda i: (i, 0)),
        out_shape=jax.ShapeDtypeStruct((N, 40), F32),
    )(t3, y3, dinv, b3)


def kernel(x, edge_index, W1, b1, W2, b2, W3, b3):
    src = edge_index[0].reshape(NW, NCH, CH)
    dst = edge_index[1].reshape(NW, NCH, CH)
    W3p = jnp.pad(W3, ((0, 0), (0, 88)))

    degp = _make_deg()(dst)
    dinv, y10, y11 = _ka(degp, x)

    t10, t11 = _make_prop(2)(y10, y11, src, dst)
    y2s = _k1(t10, t11, y10, y11, dinv, W1, b1.reshape(1, -1), W2)

    t2s = _make_prop(4)(*y2s, src, dst)
    y3 = _k3(t2s, y2s, dinv, b2.reshape(1, -1), W3p)

    (t3,) = _make_prop(1)(y3, src, dst)
    return _k4(t3, y3, dinv, b3.reshape(1, -1))


# speedup vs baseline: 1.0877x; 1.0011x over previous
import functools

import jax
import jax.numpy as jnp
from jax import lax
from jax.experimental import pallas as pl
from jax.experimental.pallas import tpu as pltpu
from jax.experimental.pallas import tpu_sc as plsc

N = 10000
E = 160000
NCORE = 2
NSUB = 16
NW = NCORE * NSUB
EPW = E // NW
CH = 125
NCH = EPW // CH
NPAD = 10240
RPS = NPAD // NSUB
RZ = 128
RB = 2000

F32 = jnp.float32


def _fill(ref, rows, cols, value):
    per_row = cols // 16

    def body(i, carry):
        r = i // per_row
        c = (i % per_row) * 16
        ref[r, pl.ds(c, 16)] = jnp.full((16,), value, F32)
        return carry

    lax.fori_loop(0, rows * per_row, body, 0)


def _deg_body(dst_hbm, out_hbm, didx, ones, zbuf, acc, lsem):
    cid = lax.axis_index("c")
    sid = lax.axis_index("s")
    wid = cid * NSUB + sid
    _fill(ones, CH, 128, 1.0)
    _fill(zbuf, RZ, 128, 0.0)
    pltpu.sync_copy(dst_hbm.at[wid], didx)
    dz = [pltpu.async_copy(zbuf, acc.at[pl.ds(sid * RPS + z * RZ, RZ)], lsem)
          for z in range(RPS // RZ)]
    for d in dz:
        d.wait()
    plsc.subcore_barrier()
    for c in range(NCH):
        pltpu.sync_copy(ones, acc.at[didx.at[c]], add=True)
    plsc.subcore_barrier()
    dd = [pltpu.async_copy(acc.at[pl.ds(sid * RPS + z * RZ, RZ)],
                           out_hbm.at[cid, pl.ds(sid * RPS + z * RZ, RZ)], lsem)
          for z in range(RPS // RZ)]
    for d in dd:
        d.wait()


def _make_deg():
    mesh = plsc.VectorSubcoreMesh(core_axis_name="c", subcore_axis_name="s")
    return pl.kernel(
        _deg_body,
        out_type=jax.ShapeDtypeStruct((NCORE, NPAD, 128), F32),
        mesh=mesh,
        scratch_types=[
            pltpu.VMEM((NCH, CH), jnp.int32),
            pltpu.VMEM((CH, 128), F32),
            pltpu.VMEM((RZ, 128), F32),
            pltpu.VMEM_SHARED((NPAD, 128), F32),
            pltpu.SemaphoreType.DMA,
        ],
    )


def _make_prop(nb):
    mesh = plsc.VectorSubcoreMesh(core_axis_name="c", subcore_axis_name="s")
    bd = 128

    def body(*refs):
        y = refs[:nb]
        src_hbm, dst_hbm = refs[nb], refs[nb + 1]
        outs = refs[nb + 2:2 * nb + 2]
        (sidx, didx, gbuf, zbuf, acc, sem, lsem) = refs[2 * nb + 2:]
        cid = lax.axis_index("c")
        sid = lax.axis_index("s")
        wid = cid * NSUB + sid
        _fill(zbuf, RZ, bd, 0.0)
        pltpu.sync_copy(src_hbm.at[wid], sidx)
        pltpu.sync_copy(dst_hbm.at[wid], didx)
        gd = pltpu.async_copy(y[0].at[sidx.at[0]], gbuf, sem)
        for b in range(nb):
            dz = [pltpu.async_copy(zbuf,
                                   acc.at[pl.ds(sid * RPS + z * RZ, RZ)], lsem)
                  for z in range(RPS // RZ)]
            for d in dz:
                d.wait()
            plsc.subcore_barrier()
            for c in range(NCH):
                gd.wait()
                pltpu.sync_copy(gbuf, acc.at[didx.at[c]], add=True)
                if c + 1 < NCH:
                    gd = pltpu.async_copy(y[b].at[sidx.at[c + 1]], gbuf, sem)
                elif b + 1 < nb:
                    gd = pltpu.async_copy(y[b + 1].at[sidx.at[0]], gbuf, sem)
            plsc.subcore_barrier()
            dd = [pltpu.async_copy(acc.at[pl.ds(sid * RPS + z * RZ, RZ)],
                                   outs[b].at[cid,
                                              pl.ds(sid * RPS + z * RZ, RZ)],
                                   lsem)
                  for z in range(RPS // RZ)]
            for d in dd:
                d.wait()

    return pl.kernel(
        body,
        out_type=[jax.ShapeDtypeStruct((NCORE, NPAD, bd), F32) for _ in range(nb)],
        mesh=mesh,
        scratch_types=[
            pltpu.VMEM((NCH, CH), jnp.int32),
            pltpu.VMEM((NCH, CH), jnp.int32),
            pltpu.VMEM((CH, bd), F32),
            pltpu.VMEM((RZ, bd), F32),
            pltpu.VMEM_SHARED((NPAD, bd), F32),
            pltpu.SemaphoreType.DMA,
            pltpu.SemaphoreType.DMA,
        ],
    )


def _mm(a, b):
    return lax.dot_general(a, b, (((1,), (0,)), ((), ())),
                           precision=lax.Precision.DEFAULT,
                           preferred_element_type=F32)


def _ka_body(degp, x, dinv_o, y0_o, y1_o):
    deg = degp[0, :, 0:1] + degp[1, :, 0:1] + 1.0
    dv = lax.rsqrt(deg)
    dinv_o[...] = dv
    y = x[...] * dv
    y0_o[...] = y[:, :128]
    y1_o[...] = y[:, 128:]


def _ka(degp, x):
    return pl.pallas_call(
        _ka_body,
        grid=(N // RB,),
        in_specs=[
            pl.BlockSpec((NCORE, RB, 128), lambda i: (0, i, 0)),
            pl.BlockSpec((RB, 256), lambda i: (i, 0)),
        ],
        out_specs=[
            pl.BlockSpec((RB, 1), lambda i: (i, 0)),
            pl.BlockSpec((RB, 128), lambda i: (i, 0)),
            pl.BlockSpec((RB, 128), lambda i: (i, 0)),
        ],
        out_shape=[
            jax.ShapeDtypeStruct((N, 1), F32),
            jax.ShapeDtypeStruct((N, 128), F32),
            jax.ShapeDtypeStruct((N, 128), F32),
        ],
    )(degp, x)


def _k1_body(t10, t11, y10, y11, dinv, W1, b1, W2, o0, o1, o2, o3):
    dv = dinv[...]
    p0 = dv * (t10[0] + t10[1] + y10[...])
    p1 = dv * (t11[0] + t11[1] + y11[...])
    W1m = W1[...]
    h = _mm(p0, W1m[:128]) + _mm(p1, W1m[128:]) + b1[...]
    h = jnp.maximum(h, 0.0)
    y2 = _mm(h, W2[...]) * dv
    o0[...] = y2[:, 0:128]
    o1[...] = y2[:, 128:256]
    o2[...] = y2[:, 256:384]
    o3[...] = y2[:, 384:512]


def _k1(t10, t11, y10, y11, dinv, W1, b1, W2):
    tspec = pl.BlockSpec((NCORE, RB, 128), lambda i: (0, i, 0))
    yspec = pl.BlockSpec((RB, 128), lambda i: (i, 0))
    return pl.pallas_call(
        _k1_body,
        grid=(N // RB,),
        in_specs=[
            tspec, tspec, yspec, yspec,
            pl.BlockSpec((RB, 1), lambda i: (i, 0)),
            pl.BlockSpec((256, 512), lambda i: (0, 0)),
            pl.BlockSpec((1, 512), lambda i: (0, 0)),
            pl.BlockSpec((512, 512), lambda i: (0, 0)),
        ],
        out_specs=[yspec, yspec, yspec, yspec],
        out_shape=[jax.ShapeDtypeStruct((N, 128), F32) for _ in range(4)],
    )(t10, t11, y10, y11, dinv, W1, b1, W2)


def _k3_body(t20, t21, t22, t23, y20, y21, y22, y23, dinv, b2, W3p, out):
    dv = dinv[...]
    W3m = W3p[...]
    b2m = b2[...]
    ts = (t20, t21, t22, t23)
    ys = (y20, y21, y22, y23)
    z = jnp.zeros((RB, 128), F32)
    for b in range(4):
        q = ts[b][0] + ts[b][1] + ys[b][...]
        h = jnp.maximum(dv * q + b2m[:, b * 128:(b + 1) * 128], 0.0)
        z = z + _mm(h, W3m[b * 128:(b + 1) * 128])
    out[...] = z * dv


def _k3(t2s, y2s, dinv, b2, W3p):
    tspec = pl.BlockSpec((NCORE, RB, 128), lambda i: (0, i, 0))
    yspec = pl.BlockSpec((RB, 128), lambda i: (i, 0))
    return pl.pallas_call(
        _k3_body,
        grid=(N // RB,),
        in_specs=[
            tspec, tspec, tspec, tspec,
            yspec, yspec, yspec, yspec,
            pl.BlockSpec((RB, 1), lambda i: (i, 0)),
            pl.BlockSpec((1, 512), lambda i: (0, 0)),
            pl.BlockSpec((512, 128), lambda i: (0, 0)),
        ],
        out_specs=pl.BlockSpec((RB, 128), lambda i: (i, 0)),
        out_shape=jax.ShapeDtypeStruct((N, 128), F32),
    )(*t2s, *y2s, dinv, b2, W3p)


def _k4_body(t3, y3, dinv, b3, out):
    q = t3[0] + t3[1] + y3[...]
    out[...] = dinv[...] * q[:, :40] + b3[...]


def _k4(t3, y3, dinv, b3):
    return pl.pallas_call(
        _k4_body,
        grid=(N // RB,),
        in_specs=[
            pl.BlockSpec((NCORE, RB, 128), lambda i: (0, i, 0)),
            pl.BlockSpec((RB, 128), lambda i: (i, 0)),
            pl.BlockSpec((RB, 1), lambda i: (i, 0)),
            pl.BlockSpec((1, 40), lambda i: (0, 0)),
        ],
        out_specs=pl.BlockSpec((RB, 40), lambda i: (i, 0)),
        out_shape=jax.ShapeDtypeStruct((N, 40), F32),
    )(t3, y3, dinv, b3)


def kernel(x, edge_index, W1, b1, W2, b2, W3, b3):
    src = edge_index[0].reshape(NW, NCH, CH)
    dst = edge_index[1].reshape(NW, NCH, CH)
    W3p = jnp.pad(W3, ((0, 0), (0, 88)))

    degp = _make_deg()(dst)
    dinv, y10, y11 = _ka(degp, x)

    t10, t11 = _make_prop(2)(y10, y11, src, dst)
    y2s = _k1(t10, t11, y10, y11, dinv, W1, b1.reshape(1, -1), W2)

    t2s = _make_prop(4)(*y2s, src, dst)
    y3 = _k3(t2s, y2s, dinv, b2.reshape(1, -1), W3p)

    (t3,) = _make_prop(1)(y3, src, dst)
    return _k4(t3, y3, dinv, b3.reshape(1, -1))
